# baseline (speedup 1.0000x reference)
import functools

import jax
import jax.numpy as jnp
from jax.experimental import pallas as pl
from jax.experimental.pallas import tpu as pltpu
from jax.experimental.pallas import tpu_sc as plsc

N_WAY = 200
N_SUP = 1000
SPAD = 1024
N_Q = 4096
N_SAMP = 8
FEAT = 2048
BQ = 256
BFQ = BQ * N_SAMP
BS = 128
BFS = BS * N_SAMP
NSUP_STEPS = SPAD // BS
NQ_STEPS = N_Q // BQ
CPAD = 256


NSC_WORKERS = 16
SC_CHUNK = SPAD // NSC_WORKERS


def _make_sc_counts():
    mesh = plsc.VectorSubcoreMesh(core_axis_name="c", subcore_axis_name="s")

    @functools.partial(
        pl.kernel, mesh=mesh,
        out_type=jax.ShapeDtypeStruct((CPAD,), jnp.float32),
        scratch_types=[
            pltpu.VMEM((SC_CHUNK,), jnp.int32),
            pltpu.VMEM((SC_CHUNK,), jnp.float32),
            pltpu.VMEM((CPAD,), jnp.float32),
            pltpu.VMEM_SHARED((CPAD,), jnp.float32),
        ],
    )
    def sc_counts(lab_hbm, out_hbm, idx_v, ones_v, cnt_v, bins_sh):
        cid = jax.lax.axis_index("c")
        sid = jax.lax.axis_index("s")

        @pl.when(jnp.logical_and(cid == 0, sid == 0))
        def _zero_bins():
            for j in range(CPAD // 16):
                cnt_v[pl.ds(j * 16, 16)] = jnp.zeros((16,), jnp.float32)
            pltpu.sync_copy(cnt_v, bins_sh)

        plsc.subcore_barrier()

        @pl.when(cid == 0)
        def _scatter():
            for j in range(SC_CHUNK // 16):
                ones_v[pl.ds(j * 16, 16)] = jnp.ones((16,), jnp.float32)
            pltpu.sync_copy(lab_hbm.at[pl.ds(sid * SC_CHUNK, SC_CHUNK)],
                            idx_v)
            pltpu.sync_copy(ones_v, bins_sh.at[idx_v], add=True)

        plsc.subcore_barrier()

        @pl.when(jnp.logical_and(cid == 0, sid == 0))
        def _publish():
            pltpu.sync_copy(bins_sh, cnt_v)
            pltpu.sync_copy(cnt_v, out_hbm)

    return sc_counts


def _fused_kernel(supf_ref, qfa_ref, qfb_ref, lab_ref, cnt_ref, out_ref,
                  sel_scr, oh_scr, scale_scr, rhs_scr):
    i = pl.program_id(0)

    @pl.when(i == 0)
    def _init():
        r = jax.lax.broadcasted_iota(jnp.int32, (BQ, BFQ), 1)
        c = jax.lax.broadcasted_iota(jnp.int32, (BQ, BFQ), 0)
        sel_scr[...] = jnp.where(r // N_SAMP == c, 0.125, 0.0
                                 ).astype(jnp.bfloat16)
        lab = lab_ref[...]
        cls = jax.lax.broadcasted_iota(jnp.int32, (SPAD, CPAD), 1)
        oh = lab == cls
        oh_scr[...] = oh.astype(jnp.bfloat16)
        counts = cnt_ref[...]
        scale_scr[...] = jnp.where(counts > 0, -1.0 / counts, 0.0)

    def _support_body(supf):
        smf = jax.lax.dot_general(
            sel_scr[0:BS, 0:BFS], supf, (((1,), (0,)), ((), ())),
            preferred_element_type=jnp.float32)
        s2 = jnp.sum(smf * smf, axis=1, keepdims=True)
        rhs_scr[pl.ds(i * BS, BS), :] = jnp.concatenate(
            [(-2.0 * smf).astype(jnp.bfloat16), s2.astype(jnp.bfloat16)],
            axis=1)

    @pl.when(i < NSUP_STEPS - 1)
    def _support():
        _support_body(supf_ref[...].astype(jnp.bfloat16))

    @pl.when(i == NSUP_STEPS - 1)
    def _support_last():
        flat = i * BFS + jax.lax.broadcasted_iota(jnp.int32, (BFS, 1), 0)
        _support_body(jnp.where(flat < N_SUP * N_SAMP, supf_ref[...], 0.0
                                ).astype(jnp.bfloat16))

    @pl.when(i >= NSUP_STEPS)
    def _query():
        top = qfa_ref[...].astype(jnp.bfloat16)
        bot = qfb_ref[...].astype(jnp.bfloat16)
        qm = (jax.lax.dot_general(
                  sel_scr[:, 0:BFQ // 2], top, (((1,), (0,)), ((), ())),
                  preferred_element_type=jnp.float32)
              + jax.lax.dot_general(
                  sel_scr[:, BFQ // 2:BFQ], bot, (((1,), (0,)), ((), ())),
                  preferred_element_type=jnp.float32))
        q2 = jnp.sum(qm * qm, axis=1, keepdims=True)
        lhs = jnp.concatenate(
            [qm.astype(jnp.bfloat16), jnp.ones((BQ, 1), jnp.bfloat16)],
            axis=1)
        dots = jax.lax.dot_general(
            lhs, rhs_scr[...], (((1,), (1,)), ((), ())),
            preferred_element_type=jnp.float32)
        dist = jnp.sqrt(jnp.maximum(q2 + dots, 1e-12)).astype(jnp.bfloat16)
        sums = jax.lax.dot_general(
            dist, oh_scr[...], (((1,), (0,)), ((), ())),
            preferred_element_type=jnp.float32)
        out_ref[...] = (sums * scale_scr[...])[:, :N_WAY]


def kernel(support_set, support_labels, queries):
    supf = support_set.reshape(N_SUP * N_SAMP, FEAT)
    qf = queries.reshape(N_Q * N_SAMP, FEAT)
    lab_col = jnp.concatenate(
        [support_labels.astype(jnp.int32),
         jnp.full((SPAD - N_SUP,), -1, jnp.int32)]).reshape(SPAD, 1)
    lab_sc = jnp.concatenate(
        [support_labels.astype(jnp.int32),
         jnp.full((SPAD - N_SUP,), CPAD - 1, jnp.int32)])
    counts = _make_sc_counts()(lab_sc).reshape(1, CPAD)
    nsup_blocks = (N_SUP * N_SAMP + BFS - 1) // BFS
    out = pl.pallas_call(
        _fused_kernel,
        grid=(NSUP_STEPS + NQ_STEPS,),
        in_specs=[
            pl.BlockSpec((BFS, FEAT),
                         lambda i: (jnp.minimum(i, nsup_blocks - 1), 0)),
            pl.BlockSpec((BFQ // 2, FEAT),
                         lambda i: (2 * jnp.maximum(i - NSUP_STEPS, 0), 0)),
            pl.BlockSpec((BFQ // 2, FEAT),
                         lambda i: (2 * jnp.maximum(i - NSUP_STEPS, 0) + 1, 0)),
            pl.BlockSpec((SPAD, 1), lambda i: (0, 0)),
            pl.BlockSpec((1, CPAD), lambda i: (0, 0)),
        ],
        out_specs=pl.BlockSpec((BQ, N_WAY),
                               lambda i: (jnp.maximum(i - NSUP_STEPS, 0), 0)),
        out_shape=jax.ShapeDtypeStruct((N_Q, N_WAY), jnp.float32),
        scratch_shapes=[
            pltpu.VMEM((BQ, BFQ), jnp.bfloat16),
            pltpu.VMEM((SPAD, CPAD), jnp.bfloat16),
            pltpu.VMEM((1, CPAD), jnp.float32),
            pltpu.VMEM((SPAD, FEAT + 1), jnp.bfloat16),
        ],
        compiler_params=pltpu.CompilerParams(
            dimension_semantics=("arbitrary",)),
    )(supf, qf, qf, lab_col, counts)
    return out


# speedup vs baseline: 1.1782x; 1.1782x over previous
import jax
import jax.numpy as jnp
from jax.experimental import pallas as pl
from jax.experimental.pallas import tpu as pltpu

N_WAY = 200
N_SUP = 1000
SPAD = 1024
N_Q = 4096
N_SAMP = 8
FEAT = 2048
BQ = 256
BFQ = BQ * N_SAMP
BS = 128
BFS = BS * N_SAMP
NSUP_STEPS = SPAD // BS
NQ_STEPS = N_Q // BQ
CPAD = 256


def _fused_kernel(supf_ref, qfa_ref, qfb_ref, lab_ref, out_ref,
                  sel_scr, oh_scr, scale_scr, rhs_scr):
    i = pl.program_id(0)

    @pl.when(i == 0)
    def _init():
        r = jax.lax.broadcasted_iota(jnp.int32, (BQ, BFQ), 1)
        c = jax.lax.broadcasted_iota(jnp.int32, (BQ, BFQ), 0)
        sel_scr[...] = jnp.where(r // N_SAMP == c, 0.125, 0.0
                                 ).astype(jnp.bfloat16)
        lab = lab_ref[...]
        cls = jax.lax.broadcasted_iota(jnp.int32, (SPAD, CPAD), 1)
        oh = lab == cls
        oh_scr[...] = oh.astype(jnp.bfloat16)
        counts = jnp.sum(oh.astype(jnp.float32), axis=0, keepdims=True)
        scale_scr[...] = jnp.where(counts > 0, -1.0 / counts, 0.0)

    def _support_body(supf):
        smf = jax.lax.dot_general(
            sel_scr[0:BS, 0:BFS], supf, (((1,), (0,)), ((), ())),
            preferred_element_type=jnp.float32)
        s2 = jnp.sum(smf * smf, axis=1, keepdims=True)
        rhs_scr[pl.ds(i * BS, BS), :] = jnp.concatenate(
            [(-2.0 * smf).astype(jnp.bfloat16), s2.astype(jnp.bfloat16)],
            axis=1)

    @pl.when(i < NSUP_STEPS - 1)
    def _support():
        _support_body(supf_ref[...].astype(jnp.bfloat16))

    @pl.when(i == NSUP_STEPS - 1)
    def _support_last():
        flat = i * BFS + jax.lax.broadcasted_iota(jnp.int32, (BFS, 1), 0)
        _support_body(jnp.where(flat < N_SUP * N_SAMP, supf_ref[...], 0.0
                                ).astype(jnp.bfloat16))

    @pl.when(i >= NSUP_STEPS)
    def _query():
        top = qfa_ref[...].astype(jnp.bfloat16)
        bot = qfb_ref[...].astype(jnp.bfloat16)
        qm = (jax.lax.dot_general(
                  sel_scr[:, 0:BFQ // 2], top, (((1,), (0,)), ((), ())),
                  preferred_element_type=jnp.float32)
              + jax.lax.dot_general(
                  sel_scr[:, BFQ // 2:BFQ], bot, (((1,), (0,)), ((), ())),
                  preferred_element_type=jnp.float32))
        q2 = jnp.sum(qm * qm, axis=1, keepdims=True)
        lhs = jnp.concatenate(
            [qm.astype(jnp.bfloat16), jnp.ones((BQ, 1), jnp.bfloat16)],
            axis=1)
        dots = jax.lax.dot_general(
            lhs, rhs_scr[...], (((1,), (1,)), ((), ())),
            preferred_element_type=jnp.float32)
        dist = jnp.sqrt(jnp.maximum(q2 + dots, 1e-12)).astype(jnp.bfloat16)
        sums = jax.lax.dot_general(
            dist, oh_scr[...], (((1,), (0,)), ((), ())),
            preferred_element_type=jnp.float32)
        out_ref[...] = (sums * scale_scr[...])[:, :N_WAY]


def kernel(support_set, support_labels, queries):
    supf = support_set.reshape(N_SUP * N_SAMP, FEAT)
    qf = queries.reshape(N_Q * N_SAMP, FEAT)
    lab_col = jnp.concatenate(
        [support_labels.astype(jnp.int32),
         jnp.full((SPAD - N_SUP,), -1, jnp.int32)]).reshape(SPAD, 1)
    nsup_blocks = (N_SUP * N_SAMP + BFS - 1) // BFS
    out = pl.pallas_call(
        _fused_kernel,
        grid=(NSUP_STEPS + NQ_STEPS,),
        in_specs=[
            pl.BlockSpec((BFS, FEAT),
                         lambda i: (jnp.minimum(i, nsup_blocks - 1), 0)),
            pl.BlockSpec((BFQ // 2, FEAT),
                         lambda i: (2 * jnp.maximum(i - NSUP_STEPS, 0), 0)),
            pl.BlockSpec((BFQ // 2, FEAT),
                         lambda i: (2 * jnp.maximum(i - NSUP_STEPS, 0) + 1, 0)),
            pl.BlockSpec((SPAD, 1), lambda i: (0, 0)),
        ],
        out_specs=pl.BlockSpec((BQ, N_WAY),
                               lambda i: (jnp.maximum(i - NSUP_STEPS, 0), 0)),
        out_shape=jax.ShapeDtypeStruct((N_Q, N_WAY), jnp.float32),
        scratch_shapes=[
            pltpu.VMEM((BQ, BFQ), jnp.bfloat16),
            pltpu.VMEM((SPAD, CPAD), jnp.bfloat16),
            pltpu.VMEM((1, CPAD), jnp.float32),
            pltpu.VMEM((SPAD, FEAT + 1), jnp.bfloat16),
        ],
        compiler_params=pltpu.CompilerParams(
            dimension_semantics=("arbitrary",)),
    )(supf, qf, qf, lab_col)
    return out
